# baseline (speedup 1.0000x reference)
import jax
import jax.numpy as jnp
from jax import lax
from jax.experimental import pallas as pl
from jax.experimental.pallas import tpu as pltpu, tpu_sc as plsc

S2 = 49
LENGTH = 30
NB = 10
COORD, NOOBJ = 5.0, 0.5

NC, NS = 2, 16
NW = NC * NS
CPC = 4
NCHUNK = 12
BW = 256
HALVES = 2
NUNIT = NCHUNK * HALVES

BWT = 1024


def _box_term(P, T):
    def boxes(V):
        out = []
        for i in range(2):
            x, y, w, h, c = (V[5 * i + k] for k in range(5))
            w2 = w * w
            h2 = h * h
            out.append((x - 0.5 * w2, y - 0.5 * h2, x + 0.5 * w2,
                        y + 0.5 * h2, c, w2 * h2))
        return out

    BP = boxes(P)
    BT = boxes(T)
    conf = T[9]

    def inter_union(bp, bt):
        ltx = jnp.maximum(bp[0], bt[0])
        lty = jnp.maximum(bp[1], bt[1])
        rbx = jnp.minimum(bp[2], bt[2])
        rby = jnp.minimum(bp[3], bt[3])
        zero = jnp.zeros_like(ltx)
        wx = jnp.maximum(rbx - ltx, zero)
        wy = jnp.maximum(rby - lty, zero)
        inter = wx * wy
        return inter, bp[5] + bt[5] - inter

    g01 = []
    for j in range(2):
        i0, u0 = inter_union(BP[0], BT[j])
        i1, u1 = inter_union(BP[1], BT[j])
        g01.append(i1 * u0 > i0 * u1)
    coord_on = conf > 0
    one = jnp.ones_like(conf)
    zero = jnp.zeros_like(conf)
    w0 = jnp.where(jnp.logical_and(jnp.logical_not(jnp.logical_and(g01[0], g01[1])),
                                   coord_on), one, zero)
    w1 = jnp.where(jnp.logical_and(jnp.logical_or(g01[0], g01[1]), coord_on),
                   one, zero)

    term = jnp.zeros_like(conf)
    for i, wgt in ((0, w0), (1, w1)):
        dx = BP[i][0] - BT[i][0]
        dy = BP[i][1] - BT[i][1]
        dX = BP[i][2] - BT[i][2]
        dY = BP[i][3] - BT[i][3]
        dc = BP[i][4] - BT[i][4]
        term = term + wgt * (COORD * (dx * dx + dy * dy + dX * dX + dY * dY)
                             + dc * dc)
    return term


def _make_sc(batch):
    mesh = plsc.VectorSubcoreMesh(core_axis_name="c", subcore_axis_name="s")

    @pl.kernel(
        out_type=jax.ShapeDtypeStruct((NW, 16), jnp.float32),
        mesh=mesh,
        compiler_params=pltpu.CompilerParams(
            needs_layout_passes=False, use_tc_tiling_on_sc=True),
        scratch_types=(
            [pltpu.VMEM((16, BW), jnp.float32)] * (2 * CPC) +
            [pltpu.VMEM((14, BW), jnp.float32)] * (2 * CPC) + [
                pltpu.VMEM((16,), jnp.float32),
                pltpu.SemaphoreType.DMA,
                pltpu.SemaphoreType.DMA,
            ]),
    )
    def k(pt_hbm, tt_hbm, out_hbm, *rest):
        bufs, (accbuf, sem0, sem1) = rest[:2 * 2 * CPC], rest[2 * 2 * CPC:]
        pbufs = (bufs[0:CPC], bufs[CPC:2 * CPC])
        tbufs = (bufs[2 * CPC:3 * CPC], bufs[3 * CPC:4 * CPC])
        sems = (sem0, sem1)
        wid = lax.axis_index("s") * NC + lax.axis_index("c")

        OFF = tuple((6 * j + 4) % 8 for j in range(CPC))

        def unit_copies(u, slot):
            b0 = wid * 512 + (u // NCHUNK) * BW
            g = u % NCHUNK
            ops = []
            for j in range(CPC):
                cell = g * CPC + j
                ops.append(pltpu.make_async_copy(
                    pt_hbm.at[pl.ds(pl.multiple_of(
                        cell * LENGTH + 20 - OFF[j], 8), 16),
                              pl.ds(b0, BW)],
                    pbufs[slot][j], sems[slot]))
                ops.append(pltpu.make_async_copy(
                    tt_hbm.at[cell, pl.ds(16, 14), pl.ds(b0, BW)],
                    tbufs[slot][j], sems[slot]))
            return ops

        for op in unit_copies(0, 0):
            op.start()

        def compute(pb, tb, acc):
            def lane_body(l, a):
                for j in range(CPC):
                    P = [pb[j][OFF[j] + c, pl.ds(l * 16, 16)]
                         for c in range(NB)]
                    T = [tb[j][4 + c, pl.ds(l * 16, 16)] for c in range(NB)]
                    a = a + _box_term(P, T)
                return a
            return lax.fori_loop(0, BW // 16, lane_body, acc)

        def pair_body(i, acc):
            for b in (0, 1):
                u = 2 * i + b

                @pl.when(u + 1 < NUNIT)
                def _():
                    for op in unit_copies(u + 1, 1 - b):
                        op.start()

                for op in unit_copies(u, b):
                    op.wait()
                acc = compute(pbufs[b], tbufs[b], acc)
            return acc

        acc = lax.fori_loop(0, NUNIT // 2, pair_body,
                            jnp.zeros((16,), jnp.float32))

        c48 = []
        for h in range(HALVES):
            b0 = wid * 512 + h * BW
            c48.append(pltpu.make_async_copy(
                pt_hbm.at[pl.ds(48 * LENGTH + 16, 14), pl.ds(b0, BW)],
                tbufs[1][h], sem0))
            c48.append(pltpu.make_async_copy(
                tt_hbm.at[48, pl.ds(16, 14), pl.ds(b0, BW)],
                tbufs[0][h], sem0))
        for op in c48:
            op.start()
        for op in c48:
            op.wait()

        def lane48(l, a):
            for h in range(HALVES):
                P = [tbufs[1][h][4 + c, pl.ds(l * 16, 16)]
                     for c in range(NB)]
                T = [tbufs[0][h][4 + c, pl.ds(l * 16, 16)]
                     for c in range(NB)]
                a = a + _box_term(P, T)
            return a

        acc = lax.fori_loop(0, BW // 16, lane48, acc)
        accbuf[...] = acc
        pltpu.sync_copy(accbuf, out_hbm.at[wid])

    return k


def _tc_class_kernel(pt_ref, tt_ref, out_ref):
    acc = jnp.zeros((1, BWT), jnp.float32)
    for g in range(S2):
        d = pt_ref[g * LENGTH:g * LENGTH + 20, :] - tt_ref[g, 0:20, :]
        sq = d * d
        s_cls = jnp.sum(sq, axis=0, keepdims=True)
        v49 = sq[4:5, :] + sq[9:10, :]
        conf = tt_ref[g, 29:30, :]
        zero = jnp.zeros_like(conf)
        acc = acc + jnp.where(conf > 0, s_cls, zero)
        acc = acc + jnp.where(conf == 0, NOOBJ * v49, zero)
    out_ref[...] = acc


def _make_tc(batch):
    grid = batch // BWT
    return pl.pallas_call(
        _tc_class_kernel,
        grid=(grid,),
        in_specs=[
            pl.BlockSpec((S2 * LENGTH, BWT), lambda b: (0, b)),
            pl.BlockSpec((S2, LENGTH, BWT), lambda b: (0, 0, b)),
        ],
        out_specs=pl.BlockSpec((1, BWT), lambda b: (0, b)),
        out_shape=jax.ShapeDtypeStruct((1, batch), jnp.float32),
    )


def kernel(predict, target):
    batch = target.shape[0]
    pt = predict.T
    tt = jnp.transpose(target, (1, 2, 3, 0)).reshape(S2, LENGTH, batch)
    box_partials = _make_sc(batch)(pt, tt)
    cls_partials = _make_tc(batch)(pt, tt)
    return jnp.sum(box_partials) + jnp.sum(cls_partials)


# speedup vs baseline: 1.2004x; 1.2004x over previous
import jax
import jax.numpy as jnp
from jax import lax
from jax.experimental import pallas as pl
from jax.experimental.pallas import tpu as pltpu, tpu_sc as plsc

S2 = 49
LENGTH = 30
COORD, NOOBJ = 5.0, 0.5

NC, NS = 2, 16
NW = NC * NS
CPC = 4
NCHUNK = 12
BW = 256
HALVES = 2
NUNIT = NCHUNK * HALVES


def _cell_term(P, T):
    d4 = P[4] - T[4]
    d9 = P[9] - T[9]
    s_cls = d4 * d4 + d9 * d9
    v49 = s_cls
    for c in range(20):
        if c in (4, 9):
            continue
        d = P[c] - T[c]
        s_cls = s_cls + d * d

    def boxes(V):
        out = []
        for i in range(2):
            x, y, w, h, c = (V[20 + 5 * i + k] for k in range(5))
            w2 = w * w
            h2 = h * h
            out.append((x - 0.5 * w2, y - 0.5 * h2, x + 0.5 * w2,
                        y + 0.5 * h2, c, w2 * h2))
        return out

    BP = boxes(P)
    BT = boxes(T)
    conf = T[29]

    def inter_union(bp, bt):
        ltx = jnp.maximum(bp[0], bt[0])
        lty = jnp.maximum(bp[1], bt[1])
        rbx = jnp.minimum(bp[2], bt[2])
        rby = jnp.minimum(bp[3], bt[3])
        zero = jnp.zeros_like(ltx)
        wx = jnp.maximum(rbx - ltx, zero)
        wy = jnp.maximum(rby - lty, zero)
        inter = wx * wy
        return inter, bp[5] + bt[5] - inter

    g01 = []
    for j in range(2):
        i0, u0 = inter_union(BP[0], BT[j])
        i1, u1 = inter_union(BP[1], BT[j])
        g01.append(i1 * u0 > i0 * u1)
    coord_on = conf > 0
    one = jnp.ones_like(conf)
    zero = jnp.zeros_like(conf)
    w0 = jnp.where(jnp.logical_and(jnp.logical_not(jnp.logical_and(g01[0], g01[1])),
                                   coord_on), one, zero)
    w1 = jnp.where(jnp.logical_and(jnp.logical_or(g01[0], g01[1]), coord_on),
                   one, zero)

    term = jnp.where(coord_on, s_cls, zero)
    term = term + jnp.where(conf == 0, NOOBJ * v49, zero)
    for i, wgt in ((0, w0), (1, w1)):
        dx = BP[i][0] - BT[i][0]
        dy = BP[i][1] - BT[i][1]
        dX = BP[i][2] - BT[i][2]
        dY = BP[i][3] - BT[i][3]
        dc = BP[i][4] - BT[i][4]
        term = term + wgt * (COORD * (dx * dx + dy * dy + dX * dX + dY * dY)
                             + dc * dc)
    return term


def _make(batch):
    mesh = plsc.VectorSubcoreMesh(core_axis_name="c", subcore_axis_name="s")

    @pl.kernel(
        out_type=jax.ShapeDtypeStruct((NW, 16), jnp.float32),
        mesh=mesh,
        compiler_params=pltpu.CompilerParams(
            needs_layout_passes=False, use_tc_tiling_on_sc=True),
        scratch_types=[
            pltpu.VMEM((CPC * LENGTH, BW), jnp.float32),
            pltpu.VMEM((CPC * LENGTH, BW), jnp.float32),
            pltpu.VMEM((CPC, LENGTH, BW), jnp.float32),
            pltpu.VMEM((CPC, LENGTH, BW), jnp.float32),
            pltpu.VMEM((16,), jnp.float32),
            pltpu.SemaphoreType.DMA,
            pltpu.SemaphoreType.DMA,
        ],
    )
    def k(pt_hbm, tt_hbm, out_hbm, pb0, pb1, tb0, tb1, accbuf, sem0, sem1):
        wid = lax.axis_index("s") * NC + lax.axis_index("c")
        pbufs, tbufs, sems = (pb0, pb1), (tb0, tb1), (sem0, sem1)

        def unit_copies(u, slot):
            b0 = wid * 512 + (u // NCHUNK) * BW
            g = u % NCHUNK
            return [
                pltpu.make_async_copy(
                    pt_hbm.at[pl.ds(g * (CPC * LENGTH), CPC * LENGTH),
                              pl.ds(b0, BW)], pbufs[slot], sems[slot]),
                pltpu.make_async_copy(
                    tt_hbm.at[pl.ds(g * CPC, CPC), :, pl.ds(b0, BW)],
                    tbufs[slot], sems[slot]),
            ]

        for op in unit_copies(0, 0):
            op.start()

        def compute(pbuf, tbuf, acc):
            def lane_body(l, a):
                for j in range(CPC):
                    P = [pbuf[j * LENGTH + c, pl.ds(l * 16, 16)]
                         for c in range(LENGTH)]
                    T = [tbuf[j, c, pl.ds(l * 16, 16)]
                         for c in range(LENGTH)]
                    a = a + _cell_term(P, T)
                return a
            return lax.fori_loop(0, BW // 16, lane_body, acc)

        def pair_body(i, acc):
            for b in (0, 1):
                u = 2 * i + b

                @pl.when(u + 1 < NUNIT)
                def _():
                    for op in unit_copies(u + 1, 1 - b):
                        op.start()

                for op in unit_copies(u, b):
                    op.wait()
                acc = compute(pbufs[b], tbufs[b], acc)
            return acc

        acc = lax.fori_loop(0, NUNIT // 2, pair_body,
                            jnp.zeros((16,), jnp.float32))

        c48 = []
        for h in range(HALVES):
            b0 = wid * 512 + h * BW
            c48.append(pltpu.make_async_copy(
                pt_hbm.at[pl.ds(NCHUNK * CPC * LENGTH, LENGTH),
                          pl.ds(b0, BW)], tb0.at[h], sem0))
            c48.append(pltpu.make_async_copy(
                tt_hbm.at[NCHUNK * CPC, :, pl.ds(b0, BW)],
                tb1.at[h], sem1))
        for op in c48:
            op.start()
        for op in c48:
            op.wait()

        def lane48(l, a):
            for h in range(HALVES):
                P = [tb0[h, c, pl.ds(l * 16, 16)] for c in range(LENGTH)]
                T = [tb1[h, c, pl.ds(l * 16, 16)] for c in range(LENGTH)]
                a = a + _cell_term(P, T)
            return a

        acc = lax.fori_loop(0, 16, lane48, acc)
        accbuf[...] = acc
        pltpu.sync_copy(accbuf, out_hbm.at[wid])

    return k


def kernel(predict, target):
    batch = target.shape[0]
    pt = predict.T
    tt = jnp.transpose(target, (1, 2, 3, 0)).reshape(S2, LENGTH, batch)
    partials = _make(batch)(pt, tt)
    return jnp.sum(partials)
